# Initial kernel scaffold; baseline (speedup 1.0000x reference)
import jax
import jax.numpy as jnp
from jax.experimental import pallas as pl


def kernel(x, edge_index, params, eps):
    raise NotImplementedError("write your pallas kernel here")


# speedup vs baseline: 4.4054x; 4.4054x over previous
import functools

import jax
import jax.numpy as jnp
from jax import lax
from jax.experimental import pallas as pl
from jax.experimental.pallas import tpu as pltpu
from jax.experimental.pallas import tpu_sc as plsc

N = 10000
F = 128
E = 320000
NC = 2
NS = 16
NW = NC * NS
CH = 128
NCH = 79
EP = NW * NCH * CH
NP = 10240
NRS = NP // NS
ZB = 64
NZ = NRS // ZB
BN = 2000
G = N // BN

_mesh = plsc.VectorSubcoreMesh(core_axis_name="c", subcore_axis_name="s")


@functools.partial(
    pl.kernel,
    mesh=_mesh,
    out_type=[jax.ShapeDtypeStruct((NC, NP, F), jnp.float32)],
    scratch_types=[
        pltpu.VMEM((CH,), jnp.int32),
        pltpu.VMEM((CH,), jnp.int32),
        pltpu.VMEM((CH, F), jnp.float32),
        pltpu.VMEM((ZB, F), jnp.float32),
        pltpu.VMEM_SHARED((NP, F), jnp.float32),
        pltpu.SemaphoreType.DMA,
    ],
)
def _sc_agg(y_h, src_h, dst_h, z0_h, out_h, src_v, dst_v, rows_v, zb_v, acc, sem):
    c = lax.axis_index("c")
    s = lax.axis_index("s")
    w = c * NS + s
    base = w * (NCH * CH)
    pltpu.sync_copy(z0_h, zb_v)
    for k in range(NZ):
        pltpu.sync_copy(zb_v, acc.at[pl.ds(s * NRS + k * ZB, ZB)])
    plsc.subcore_barrier()

    def chunk(j, carry):
        off = base + j * CH
        pltpu.sync_copy(src_h.at[pl.ds(off, CH)], src_v)
        pltpu.sync_copy(dst_h.at[pl.ds(off, CH)], dst_v)
        pltpu.async_copy(y_h.at[src_v], rows_v, sem).wait()
        pltpu.sync_copy(rows_v, acc.at[dst_v], add=True)
        return carry

    lax.fori_loop(0, NCH, chunk, 0)
    plsc.subcore_barrier()
    for k in range(NZ):
        pltpu.sync_copy(acc.at[pl.ds(s * NRS + k * ZB, ZB)], zb_v)
        pltpu.sync_copy(zb_v, out_h.at[c, pl.ds(s * NRS + k * ZB, ZB)])


@functools.partial(
    pl.kernel,
    mesh=_mesh,
    out_type=[jax.ShapeDtypeStruct((NC, NP, F), jnp.float32)],
    scratch_types=[
        pltpu.VMEM((CH,), jnp.int32),
        pltpu.VMEM((CH, F), jnp.float32),
        pltpu.VMEM((ZB, F), jnp.float32),
        pltpu.VMEM_SHARED((NP, F), jnp.float32),
    ],
)
def _sc_deg(dst_h, z0d_h, ones_h, deg_h, dst_v, ones_v, zb_v, dacc):
    c = lax.axis_index("c")
    s = lax.axis_index("s")
    w = c * NS + s
    base = w * (NCH * CH)
    pltpu.sync_copy(z0d_h, zb_v)
    pltpu.sync_copy(ones_h, ones_v)
    for k in range(NZ):
        pltpu.sync_copy(zb_v, dacc.at[pl.ds(s * NRS + k * ZB, ZB)])
    plsc.subcore_barrier()

    def chunk(j, carry):
        pltpu.sync_copy(dst_h.at[pl.ds(base + j * CH, CH)], dst_v)
        pltpu.sync_copy(ones_v, dacc.at[dst_v], add=True)
        return carry

    lax.fori_loop(0, NCH, chunk, 0)
    plsc.subcore_barrier()
    for k in range(NZ):
        pltpu.sync_copy(dacc.at[pl.ds(s * NRS + k * ZB, ZB)], zb_v)
        pltpu.sync_copy(zb_v, deg_h.at[c, pl.ds(s * NRS + k * ZB, ZB)])


def _aggn(p_r, d_r):
    deg = d_r[0, :, 0] + d_r[1, :, 0]
    dinv = 1.0 / jnp.maximum(deg, 1.0)
    return (p_r[0] + p_r[1]) * dinv[:, None]


def _tc_gcn(h, parts, degp, Ws, Wn, b, act):
    def body(h_r, p_r, d_r, ws_r, wn_r, b_r, o_r):
        r = (jnp.dot(h_r[...], ws_r[...], preferred_element_type=jnp.float32)
             + jnp.dot(_aggn(p_r, d_r), wn_r[...],
                       preferred_element_type=jnp.float32)
             + b_r[...])
        o_r[...] = jnp.maximum(r, 0.0) if act else r

    return pl.pallas_call(
        body,
        grid=(G,),
        in_specs=[
            pl.BlockSpec((BN, F), lambda i: (i, 0)),
            pl.BlockSpec((NC, BN, F), lambda i: (0, i, 0)),
            pl.BlockSpec((NC, BN, F), lambda i: (0, i, 0)),
            pl.BlockSpec((F, F), lambda i: (0, 0)),
            pl.BlockSpec((F, F), lambda i: (0, 0)),
            pl.BlockSpec((1, F), lambda i: (0, 0)),
        ],
        out_specs=pl.BlockSpec((BN, F), lambda i: (i, 0)),
        out_shape=jax.ShapeDtypeStruct((N, F), jnp.float32),
    )(h, parts, degp, Ws, Wn, b.reshape(1, F))


def _tc_vae(h, parts, degp, wsm, wnm, bm, wsl, wnl, bl, eps):
    def body(h_r, p_r, d_r, wsm_r, wnm_r, bm_r, wsl_r, wnl_r, bl_r, e_r,
             z_r, kl_r):
        hh = h_r[...]
        an = _aggn(p_r, d_r)
        mu = (jnp.dot(hh, wsm_r[...], preferred_element_type=jnp.float32)
              + jnp.dot(an, wnm_r[...], preferred_element_type=jnp.float32)
              + bm_r[...])
        ls = (jnp.dot(hh, wsl_r[...], preferred_element_type=jnp.float32)
              + jnp.dot(an, wnl_r[...], preferred_element_type=jnp.float32)
              + bl_r[...])
        ls = jnp.minimum(ls, 10.0)
        ex = jnp.exp(ls)
        z_r[...] = mu + e_r[...] * ex
        part = jnp.sum(1.0 + 2.0 * ls - mu * mu - ex * ex)

        @pl.when(pl.program_id(0) == 0)
        def _():
            kl_r[...] = jnp.zeros((1, F), jnp.float32)

        kl_r[...] += jnp.full((1, F), part / F, jnp.float32)

    return pl.pallas_call(
        body,
        grid=(G,),
        in_specs=[
            pl.BlockSpec((BN, F), lambda i: (i, 0)),
            pl.BlockSpec((NC, BN, F), lambda i: (0, i, 0)),
            pl.BlockSpec((NC, BN, F), lambda i: (0, i, 0)),
            pl.BlockSpec((F, F), lambda i: (0, 0)),
            pl.BlockSpec((F, F), lambda i: (0, 0)),
            pl.BlockSpec((1, F), lambda i: (0, 0)),
            pl.BlockSpec((F, F), lambda i: (0, 0)),
            pl.BlockSpec((F, F), lambda i: (0, 0)),
            pl.BlockSpec((1, F), lambda i: (0, 0)),
            pl.BlockSpec((BN, F), lambda i: (i, 0)),
        ],
        out_specs=[
            pl.BlockSpec((BN, F), lambda i: (i, 0)),
            pl.BlockSpec((1, F), lambda i: (0, 0)),
        ],
        out_shape=[
            jax.ShapeDtypeStruct((N, F), jnp.float32),
            jax.ShapeDtypeStruct((1, F), jnp.float32),
        ],
    )(h, parts, degp, wsm, wnm, bm.reshape(1, F),
      wsl, wnl, bl.reshape(1, F), eps)


def kernel(x, edge_index, params, eps):
    h0 = x[0]
    pad = EP - E
    srcf = jnp.concatenate(
        [edge_index[0].astype(jnp.int32), jnp.zeros((pad,), jnp.int32)])
    dstf = jnp.concatenate(
        [edge_index[1].astype(jnp.int32), jnp.full((pad,), N, jnp.int32)])

    z0 = jnp.zeros((ZB, F), jnp.float32)
    (dp,) = _sc_deg(dstf, z0, jnp.ones((CH, F), jnp.float32))
    (p0,) = _sc_agg(h0, srcf, dstf, z0)
    h1 = _tc_gcn(h0, p0, dp, params['enc1']['Ws'], params['enc1']['Wn'],
                 params['enc1']['b'], act=True)
    (p1,) = _sc_agg(h1, srcf, dstf, z0)
    z, klb = _tc_vae(h1, p1, dp,
                     params['mu']['Ws'], params['mu']['Wn'], params['mu']['b'],
                     params['logstd']['Ws'], params['logstd']['Wn'],
                     params['logstd']['b'], eps)
    kl = jnp.sum(klb) * (-0.5 / N)
    (p2,) = _sc_agg(z, srcf, dstf, z0)
    d = _tc_gcn(z, p2, dp, params['dec1']['Ws'], params['dec1']['Wn'],
                params['dec1']['b'], act=True)
    (p3,) = _sc_agg(d, srcf, dstf, z0)
    out = _tc_gcn(d, p3, dp, params['dec2']['Ws'], params['dec2']['Wn'],
                  params['dec2']['b'], act=False)
    return out[None], kl
